# baseline (speedup 1.0000x reference)
import functools

import jax
import jax.numpy as jnp
from jax import lax
from jax.experimental import pallas as pl
from jax.experimental.pallas import tpu as pltpu
from jax.experimental.pallas import tpu_sc as plsc

_NW = 32
_IDXB = 128


def _mm_body(x_ref, w_ref, o_ref):
    o_ref[...] = jnp.dot(x_ref[...], w_ref[...],
                         preferred_element_type=jnp.float32)


def _matmul(x, w, block_rows):
    m, kd = x.shape
    _, nd = w.shape
    return pl.pallas_call(
        _mm_body,
        grid=(m // block_rows,),
        in_specs=[pl.BlockSpec((block_rows, kd), lambda i: (i, 0)),
                  pl.BlockSpec((kd, nd), lambda i: (0, 0))],
        out_specs=pl.BlockSpec((block_rows, nd), lambda i: (i, 0)),
        out_shape=jax.ShapeDtypeStruct((m, nd), jnp.float32),
    )(x, w)


def _head_body(h_ref, w_ref, b_ref, o_ref):
    logits = jnp.dot(h_ref[...], w_ref[...],
                     preferred_element_type=jnp.float32) + b_ref[...]
    mx = jnp.max(logits, axis=-1, keepdims=True)
    e = jnp.exp(logits - mx)
    o_ref[...] = e / jnp.sum(e, axis=-1, keepdims=True)


def _head(h, w3, b3, block_rows):
    m, hd = h.shape
    _, nc = w3.shape
    return pl.pallas_call(
        _head_body,
        grid=(m // block_rows,),
        in_specs=[pl.BlockSpec((block_rows, hd), lambda i: (i, 0)),
                  pl.BlockSpec((hd, nc), lambda i: (0, 0)),
                  pl.BlockSpec((1, nc), lambda i: (0, 0))],
        out_specs=pl.BlockSpec((block_rows, nc), lambda i: (i, 0)),
        out_shape=jax.ShapeDtypeStruct((m, nc), jnp.float32),
    )(h, w3, b3.reshape(1, nc))


def _prep_idx_body(g_ref, o_ref, *, k, rows, pad_rows):
    lanek = jax.lax.broadcasted_iota(jnp.int32, (rows, _IDXB), 1) % k
    o_ref[:rows, :] = g_ref[...] * k + lanek
    if pad_rows:
        o_ref[rows:, :] = jnp.zeros((pad_rows, _IDXB), jnp.int32)


def _prep_idx(g, out_rows):
    n, k = g.shape
    rows = n * k // _IDXB
    return pl.pallas_call(
        functools.partial(_prep_idx_body, k=k, rows=rows,
                          pad_rows=out_rows - rows),
        in_specs=[pl.BlockSpec((rows, _IDXB), lambda: (0, 0))],
        out_specs=pl.BlockSpec((out_rows, _IDXB), lambda: (0, 0)),
        out_shape=jax.ShapeDtypeStruct((out_rows, _IDXB), jnp.int32),
    )(g.reshape(rows, _IDXB))


def _prep_w_body(x_ref, o_ref, *, k, blk, h):
    o_ref[...] = (x_ref[...].reshape(k, blk, h)
                  .transpose(1, 0, 2).reshape(blk, k * h))


def _prep_w(w, blk):
    kb, h = w.shape
    k = kb // blk
    return pl.pallas_call(
        functools.partial(_prep_w_body, k=k, blk=blk, h=h),
        in_specs=[pl.BlockSpec((kb, h), lambda: (0, 0))],
        out_specs=pl.BlockSpec((blk, k * h), lambda: (0, 0)),
        out_shape=jax.ShapeDtypeStruct((blk, k * h), jnp.float32),
    )(w)


def _gs_body(table_ref, idx_ref, bias_ref, out_ref, *scratch, k, w, gpw):
    idx_v, buf, out_v, bias_v = scratch
    cid = lax.axis_index("c")
    sid = lax.axis_index("s")
    npg = _IDXB // k
    base = (cid * (_NW // 2) + sid) * gpw
    pltpu.sync_copy(idx_ref.at[pl.ds(base, gpw)], idx_v)
    pltpu.sync_copy(bias_ref, bias_v)
    bias = [bias_v[pl.ds(c * 16, 16)] for c in range(w // 16)]
    gw = npg * w

    def body(g, carry):
        pltpu.sync_copy(table_ref.at[idx_v.at[g]], buf)
        for t in range(npg):
            a = list(bias)
            for kk in range(k):
                row = buf.at[t * k + kk]
                for c in range(w // 16):
                    a[c] = a[c] + row[pl.ds(c * 16, 16)]
            node = g * gw + t * w
            for c in range(w // 16):
                out_v[pl.ds(node + c * 16, 16)] = a[c]
        return carry

    lax.fori_loop(0, gpw, body, 0, unroll=8)

    pltpu.sync_copy(out_v, out_ref.at[pl.ds(base * gw, gpw * gw)])


def _gather_sum(table2, idx2, bias, k, npad):
    w = table2.shape[1]
    gpw = npad * k // _IDXB // _NW
    npw = gpw * _IDXB // k
    mesh = plsc.VectorSubcoreMesh(core_axis_name="c", subcore_axis_name="s")
    kfn = pl.kernel(
        functools.partial(_gs_body, k=k, w=w, gpw=gpw),
        out_type=jax.ShapeDtypeStruct((npad * w,), jnp.float32),
        mesh=mesh,
        scratch_types=(
            [pltpu.VMEM((gpw, _IDXB), jnp.int32),
             pltpu.VMEM((_IDXB, w), jnp.float32),
             pltpu.VMEM((npw * w,), jnp.float32),
             pltpu.VMEM((w,), jnp.float32)]
        ),
        compiler_params=pltpu.CompilerParams(use_tc_tiling_on_sc=False),
    )
    return kfn(table2, idx2, bias)


def kernel(X, G, W1, b1, W2, b2, W3, b3):
    n, d = X.shape
    k = G.shape[1]
    h1d, h2d = W1.shape[1], W2.shape[1]

    npad = -(-n // _NW // 8) * _NW * 8
    idx2 = _prep_idx(G, npad)

    w1r = _prep_w(W1, d)
    w2r = _prep_w(W2, h1d)

    p = _matmul(X, w1r, 1000)
    h1 = _gather_sum(p.reshape(n * k, h1d), idx2, b1, k, npad)
    h1 = h1.reshape(npad, h1d)
    q = _matmul(h1, w2r, npad // 10)
    h2 = _gather_sum(q.reshape(npad * k, h2d), idx2, b2, k, npad)
    h2 = h2.reshape(npad, h2d)
    out = _head(h2, W3, b3, npad // 10)
    return out[:n]


# speedup vs baseline: 1.3228x; 1.3228x over previous
import functools

import jax
import jax.numpy as jnp
from jax import lax
from jax.experimental import pallas as pl
from jax.experimental.pallas import tpu as pltpu
from jax.experimental.pallas import tpu_sc as plsc

_NW = 32
_IDXB = 128
_DEPTH = 4
_GA, _GB = 116, 44


def _mm_body(x_ref, w_ref, o_ref):
    o_ref[...] = jnp.dot(x_ref[...], w_ref[...],
                         preferred_element_type=jnp.float32)


def _matmul(x, w, block_rows):
    m, kd = x.shape
    _, nd = w.shape
    return pl.pallas_call(
        _mm_body,
        grid=(m // block_rows,),
        in_specs=[pl.BlockSpec((block_rows, kd), lambda i: (i, 0)),
                  pl.BlockSpec((kd, nd), lambda i: (0, 0))],
        out_specs=pl.BlockSpec((block_rows, nd), lambda i: (i, 0)),
        out_shape=jax.ShapeDtypeStruct((m, nd), jnp.float32),
    )(x, w)


def _head_body(h_ref, w_ref, b_ref, o_ref):
    logits = jnp.dot(h_ref[...], w_ref[...],
                     preferred_element_type=jnp.float32) + b_ref[...]
    mx = jnp.max(logits, axis=-1, keepdims=True)
    e = jnp.exp(logits - mx)
    o_ref[...] = e / jnp.sum(e, axis=-1, keepdims=True)


def _head(h, w3, b3, block_rows):
    m, hd = h.shape
    _, nc = w3.shape
    return pl.pallas_call(
        _head_body,
        grid=(m // block_rows,),
        in_specs=[pl.BlockSpec((block_rows, hd), lambda i: (i, 0)),
                  pl.BlockSpec((hd, nc), lambda i: (0, 0)),
                  pl.BlockSpec((1, nc), lambda i: (0, 0))],
        out_specs=pl.BlockSpec((block_rows, nc), lambda i: (i, 0)),
        out_shape=jax.ShapeDtypeStruct((m, nc), jnp.float32),
    )(h, w3, b3.reshape(1, nc))


def _prep_idx_body(g_ref, o_ref, *, k, rows, pad_rows):
    lanek = jax.lax.broadcasted_iota(jnp.int32, (rows, _IDXB), 1) % k
    o_ref[:rows, :] = g_ref[...] * k + lanek
    if pad_rows:
        o_ref[rows:, :] = jnp.zeros((pad_rows, _IDXB), jnp.int32)


def _prep_idx(g, out_rows):
    n, k = g.shape
    rows = n * k // _IDXB
    return pl.pallas_call(
        functools.partial(_prep_idx_body, k=k, rows=rows,
                          pad_rows=out_rows - rows),
        in_specs=[pl.BlockSpec((rows, _IDXB), lambda: (0, 0))],
        out_specs=pl.BlockSpec((out_rows, _IDXB), lambda: (0, 0)),
        out_shape=jax.ShapeDtypeStruct((out_rows, _IDXB), jnp.int32),
    )(g.reshape(rows, _IDXB))


def _prep_w_body(x_ref, o_ref, *, k, blk, h):
    o_ref[...] = (x_ref[...].reshape(k, blk, h)
                  .transpose(1, 0, 2).reshape(blk, k * h))


def _prep_w(w, blk):
    kb, h = w.shape
    k = kb // blk
    return pl.pallas_call(
        functools.partial(_prep_w_body, k=k, blk=blk, h=h),
        in_specs=[pl.BlockSpec((kb, h), lambda: (0, 0))],
        out_specs=pl.BlockSpec((blk, k * h), lambda: (0, 0)),
        out_shape=jax.ShapeDtypeStruct((blk, k * h), jnp.float32),
    )(w)


def _gs_body(table_ref, idx_ref, bias_ref, out_ref, *scratch, k, w):
    idx_v = scratch[0]
    bufs = scratch[1:1 + _DEPTH]
    out_v, bias_v = scratch[1 + _DEPTH], scratch[2 + _DEPTH]
    sems = scratch[3 + _DEPTH:]
    cid = lax.axis_index("c")
    sid = lax.axis_index("s")
    npg = _IDXB // k
    base = sid * (_GA + _GB) + cid * _GA
    cnt = jnp.where(cid == 0, _GA, _GB)
    pltpu.sync_copy(idx_ref.at[pl.ds(base, _GA)], idx_v)
    pltpu.sync_copy(bias_ref, bias_v)
    bias = [bias_v[pl.ds(c * 16, 16)] for c in range(w // 16)]

    def start(g, b):
        pltpu.async_copy(table_ref.at[idx_v.at[g]], bufs[b], sems[b])

    def wait(b):
        pltpu.make_async_copy(table_ref.at[idx_v.at[0]], bufs[b],
                              sems[b]).wait()

    def acc(g, b):
        buf = bufs[b]
        for t in range(npg):
            a = list(bias)
            for kk in range(k):
                row = buf.at[t * k + kk]
                for c in range(w // 16):
                    a[c] = a[c] + row[pl.ds(c * 16, 16)]
            node = (g * npg + t) * w
            for c in range(w // 16):
                out_v[pl.ds(node + c * 16, 16)] = a[c]

    for b in range(_DEPTH):
        start(b, b)

    def ring_body(q, carry):
        g = q * _DEPTH
        for b in range(_DEPTH):
            wait(b)
            acc(g + b, b)
            start(g + b + _DEPTH, b)
        return carry

    lax.fori_loop(0, cnt // _DEPTH - 1, ring_body, 0)
    g_last = cnt - _DEPTH
    for b in range(_DEPTH):
        wait(b)
        acc(g_last + b, b)

    gw = npg * w
    pltpu.sync_copy(out_v.at[pl.ds(0, _GB * gw)],
                    out_ref.at[pl.ds(base * gw, _GB * gw)])

    @pl.when(cid == 0)
    def _():
        pltpu.sync_copy(out_v.at[pl.ds(_GB * gw, (_GA - _GB) * gw)],
                        out_ref.at[pl.ds(base * gw + _GB * gw,
                                         (_GA - _GB) * gw)])


def _gather_sum(table2, idx2, bias, k, npad):
    w = table2.shape[1]
    npw = _GA * _IDXB // k
    mesh = plsc.VectorSubcoreMesh(core_axis_name="c", subcore_axis_name="s")
    kfn = pl.kernel(
        functools.partial(_gs_body, k=k, w=w),
        out_type=jax.ShapeDtypeStruct((npad * w,), jnp.float32),
        mesh=mesh,
        scratch_types=(
            [pltpu.VMEM((_GA, _IDXB), jnp.int32)]
            + [pltpu.VMEM((_IDXB, w), jnp.float32) for _ in range(_DEPTH)]
            + [pltpu.VMEM((npw * w,), jnp.float32),
               pltpu.VMEM((w,), jnp.float32)]
            + [pltpu.SemaphoreType.DMA for _ in range(_DEPTH)]
        ),
        compiler_params=pltpu.CompilerParams(use_tc_tiling_on_sc=False),
    )
    return kfn(table2, idx2, bias)


def kernel(X, G, W1, b1, W2, b2, W3, b3):
    n, d = X.shape
    k = G.shape[1]
    h1d, h2d = W1.shape[1], W2.shape[1]

    npad = -(-n // _NW // 8) * _NW * 8
    idx_rows = (_NW // 2 - 1) * (_GA + _GB) + 2 * _GA
    idx2 = _prep_idx(G, idx_rows)

    w1r = _prep_w(W1, d)
    w2r = _prep_w(W2, h1d)

    p = _matmul(X, w1r, 1000)
    h1 = _gather_sum(p.reshape(n * k, h1d), idx2, b1, k, npad)
    h1 = h1.reshape(npad, h1d)
    q = _matmul(h1, w2r, npad // 10)
    h2 = _gather_sum(q.reshape(npad * k, h2d), idx2, b2, k, npad)
    h2 = h2.reshape(npad, h2d)
    out = _head(h2, W3, b3, npad // 10)
    return out[:n]


# speedup vs baseline: 1.3331x; 1.0078x over previous
import functools

import jax
import jax.numpy as jnp
from jax import lax
from jax.experimental import pallas as pl
from jax.experimental.pallas import tpu as pltpu
from jax.experimental.pallas import tpu_sc as plsc

_NW = 32
_IDXB = 128
_DEPTH = 2
_GA, _GB = 116, 44


def _mm_body(x_ref, w_ref, o_ref):
    o_ref[...] = jnp.dot(x_ref[...], w_ref[...],
                         preferred_element_type=jnp.float32)


def _matmul(x, w, block_rows):
    m, kd = x.shape
    _, nd = w.shape
    return pl.pallas_call(
        _mm_body,
        grid=(m // block_rows,),
        in_specs=[pl.BlockSpec((block_rows, kd), lambda i: (i, 0)),
                  pl.BlockSpec((kd, nd), lambda i: (0, 0))],
        out_specs=pl.BlockSpec((block_rows, nd), lambda i: (i, 0)),
        out_shape=jax.ShapeDtypeStruct((m, nd), jnp.float32),
    )(x, w)


def _head_body(h_ref, w_ref, b_ref, o_ref):
    logits = jnp.dot(h_ref[...], w_ref[...],
                     preferred_element_type=jnp.float32) + b_ref[...]
    mx = jnp.max(logits, axis=-1, keepdims=True)
    e = jnp.exp(logits - mx)
    o_ref[...] = e / jnp.sum(e, axis=-1, keepdims=True)


def _head(h, w3, b3, block_rows):
    m, hd = h.shape
    _, nc = w3.shape
    return pl.pallas_call(
        _head_body,
        grid=(m // block_rows,),
        in_specs=[pl.BlockSpec((block_rows, hd), lambda i: (i, 0)),
                  pl.BlockSpec((hd, nc), lambda i: (0, 0)),
                  pl.BlockSpec((1, nc), lambda i: (0, 0))],
        out_specs=pl.BlockSpec((block_rows, nc), lambda i: (i, 0)),
        out_shape=jax.ShapeDtypeStruct((m, nc), jnp.float32),
    )(h, w3, b3.reshape(1, nc))


def _prep_idx_body(g_ref, o_ref, *, k, rows, pad_rows):
    lanek = jax.lax.broadcasted_iota(jnp.int32, (rows, _IDXB), 1) % k
    o_ref[:rows, :] = g_ref[...] * k + lanek
    if pad_rows:
        o_ref[rows:, :] = jnp.zeros((pad_rows, _IDXB), jnp.int32)


def _prep_idx(g, out_rows):
    n, k = g.shape
    rows = n * k // _IDXB
    return pl.pallas_call(
        functools.partial(_prep_idx_body, k=k, rows=rows,
                          pad_rows=out_rows - rows),
        in_specs=[pl.BlockSpec((rows, _IDXB), lambda: (0, 0))],
        out_specs=pl.BlockSpec((out_rows, _IDXB), lambda: (0, 0)),
        out_shape=jax.ShapeDtypeStruct((out_rows, _IDXB), jnp.int32),
    )(g.reshape(rows, _IDXB))


def _prep_w_body(x_ref, o_ref, *, k, blk, h):
    o_ref[...] = (x_ref[...].reshape(k, blk, h)
                  .transpose(1, 0, 2).reshape(blk, k * h))


def _prep_w(w, blk):
    kb, h = w.shape
    k = kb // blk
    return pl.pallas_call(
        functools.partial(_prep_w_body, k=k, blk=blk, h=h),
        in_specs=[pl.BlockSpec((kb, h), lambda: (0, 0))],
        out_specs=pl.BlockSpec((blk, k * h), lambda: (0, 0)),
        out_shape=jax.ShapeDtypeStruct((blk, k * h), jnp.float32),
    )(w)


def _gs_body(table_ref, idx_ref, bias_ref, out_ref, *scratch, k, w):
    idx_v = scratch[0]
    bufs = scratch[1:1 + _DEPTH]
    out_v, bias_v = scratch[1 + _DEPTH], scratch[2 + _DEPTH]
    sems = scratch[3 + _DEPTH:]
    cid = lax.axis_index("c")
    sid = lax.axis_index("s")
    npg = _IDXB // k
    base = sid * (_GA + _GB) + cid * _GA
    cnt = jnp.where(cid == 0, _GA, _GB)
    pltpu.sync_copy(idx_ref.at[pl.ds(base, _GA)], idx_v)
    pltpu.sync_copy(bias_ref, bias_v)
    bias = [bias_v[pl.ds(c * 16, 16)] for c in range(w // 16)]

    def start(g, b):
        pltpu.async_copy(table_ref.at[idx_v.at[g]], bufs[b], sems[b])

    def wait(b):
        pltpu.make_async_copy(table_ref.at[idx_v.at[0]], bufs[b],
                              sems[b]).wait()

    def acc(g, b):
        buf = bufs[b]
        for t in range(npg):
            a = list(bias)
            for kk in range(k):
                row = buf.at[t * k + kk]
                for c in range(w // 16):
                    a[c] = a[c] + row[pl.ds(c * 16, 16)]
            node = (g * npg + t) * w
            for c in range(w // 16):
                out_v[pl.ds(node + c * 16, 16)] = a[c]

    for b in range(_DEPTH):
        start(b, b)

    def ring_body(q, carry):
        g = q * _DEPTH
        for b in range(_DEPTH):
            wait(b)
            acc(g + b, b)
            start(g + b + _DEPTH, b)
        return carry

    lax.fori_loop(0, cnt // _DEPTH - 1, ring_body, 0)
    g_last = cnt - _DEPTH
    for b in range(_DEPTH):
        wait(b)
        acc(g_last + b, b)

    gw = npg * w
    pltpu.sync_copy(out_v.at[pl.ds(0, _GB * gw)],
                    out_ref.at[pl.ds(base * gw, _GB * gw)])

    @pl.when(cid == 0)
    def _():
        pltpu.sync_copy(out_v.at[pl.ds(_GB * gw, (_GA - _GB) * gw)],
                        out_ref.at[pl.ds(base * gw + _GB * gw,
                                         (_GA - _GB) * gw)])


def _gather_sum(table2, idx2, bias, k, npad):
    w = table2.shape[1]
    npw = _GA * _IDXB // k
    mesh = plsc.VectorSubcoreMesh(core_axis_name="c", subcore_axis_name="s")
    kfn = pl.kernel(
        functools.partial(_gs_body, k=k, w=w),
        out_type=jax.ShapeDtypeStruct((npad * w,), jnp.float32),
        mesh=mesh,
        scratch_types=(
            [pltpu.VMEM((_GA, _IDXB), jnp.int32)]
            + [pltpu.VMEM((_IDXB, w), jnp.float32) for _ in range(_DEPTH)]
            + [pltpu.VMEM((npw * w,), jnp.float32),
               pltpu.VMEM((w,), jnp.float32)]
            + [pltpu.SemaphoreType.DMA for _ in range(_DEPTH)]
        ),
        compiler_params=pltpu.CompilerParams(use_tc_tiling_on_sc=False),
    )
    return kfn(table2, idx2, bias)


def kernel(X, G, W1, b1, W2, b2, W3, b3):
    n, d = X.shape
    k = G.shape[1]
    h1d, h2d = W1.shape[1], W2.shape[1]

    npad = -(-n // _NW // 8) * _NW * 8
    idx_rows = (_NW // 2 - 1) * (_GA + _GB) + 2 * _GA
    idx2 = _prep_idx(G, idx_rows)

    w1r = _prep_w(W1, d)
    w2r = _prep_w(W2, h1d)

    p = _matmul(X, w1r, 1000)
    h1 = _gather_sum(p.reshape(n * k, h1d), idx2, b1, k, npad)
    h1 = h1.reshape(npad, h1d)
    q = _matmul(h1, w2r, npad // 10)
    h2 = _gather_sum(q.reshape(npad * k, h2d), idx2, b2, k, npad)
    h2 = h2.reshape(npad, h2d)
    out = _head(h2, W3, b3, npad // 10)
    return out[:n]
